# Initial kernel scaffold; baseline (speedup 1.0000x reference)
import jax
import jax.numpy as jnp
from jax.experimental import pallas as pl


def kernel(template_coords, confidence, W, b, gamma, beta):
    raise NotImplementedError("write your pallas kernel here")


# speedup vs baseline: 76.2687x; 76.2687x over previous
import functools

import jax
import jax.numpy as jnp
from jax import lax
from jax.experimental import pallas as pl
from jax.experimental.pallas import tpu as pltpu
from jax.experimental.pallas import tpu_sc as plsc

_B = 2
_N = 1024
_NUM_BINS = 22
_TDIM = 16
_MAX_DIST = 40.0
_INV_W = (_NUM_BINS - 1) / _MAX_DIST

_NUM_CORES = 2
_NUM_SUBCORES = 16
_LANES = 16
_NW = _NUM_CORES * _NUM_SUBCORES
_ROWS_PER_TILE = (_B * _N) // _NW
_GROUPS = _N // _LANES


def _rsqrt(v):
  i = lax.bitcast_convert_type(v, jnp.int32)
  i = 0x5F3759DF - lax.shift_right_arithmetic(i, 1)
  y = lax.bitcast_convert_type(i, jnp.float32)
  for _ in range(3):
    y = y * (1.5 - 0.5 * v * y * y)
  return y


def _body(x_h, y_h, z_h, c_h, w_h, b_h, g_h, bt_h, out_h,
          xv, yv, zv, cv, wv, bsv, gsv, btv, tab, obuf):
  cid = lax.axis_index("c")
  sid = lax.axis_index("s")
  wid = sid * _NUM_CORES + cid
  b = wid // (_NW // _B)
  row0 = (wid * _ROWS_PER_TILE) % _N

  pltpu.sync_copy(x_h.at[b], xv)
  pltpu.sync_copy(y_h.at[b], yv)
  pltpu.sync_copy(z_h.at[b], zv)
  pltpu.sync_copy(c_h.at[b], cv)
  pltpu.sync_copy(w_h, wv)
  pltpu.sync_copy(b_h, bsv)
  pltpu.sync_copy(g_h, gsv)
  pltpu.sync_copy(bt_h, btv)

  iota = lax.iota(jnp.int32, _LANES)
  iota_w = iota * _NUM_BINS

  bias = bsv[...]
  gamma = gsv[...]
  beta = btv[...]
  for k in range(_NUM_BINS):
    col = plsc.load_gather(wv, [iota_w + k])
    x = col + bias
    mean = jnp.sum(x) * (1.0 / _TDIM)
    e = x - mean
    var = jnp.sum(e * e) * (1.0 / _TDIM)
    r = _rsqrt(jnp.full((_LANES,), var + 1e-5, jnp.float32))
    tab[k, :] = jnp.maximum(e * r * gamma + beta, 0.0)

  csplats = [jnp.full((_LANES,), c, jnp.int32) for c in range(_TDIM)]

  def row_step(kr, _):
    i = row0 + kr
    isplat = jnp.full((_LANES,), i, jnp.int32)
    xi = plsc.load_gather(xv, [isplat])
    yi = plsc.load_gather(yv, [isplat])
    zi = plsc.load_gather(zv, [isplat])
    ci = plsc.load_gather(cv, [isplat])

    def group_step(g, _):
      jb = g * _LANES
      dx = xv[pl.ds(jb, _LANES)] - xi
      dy = yv[pl.ds(jb, _LANES)] - yi
      dz = zv[pl.ds(jb, _LANES)] - zi
      s = dx * dx + dy * dy + dz * dz + 1e-8
      d = s * _rsqrt(s)
      t = jnp.minimum(d * _INV_W, float(_NUM_BINS))
      ti = t.astype(jnp.int32)
      binv = jnp.where(ti.astype(jnp.float32) < t, ti + 1, ti)
      binv = jnp.minimum(binv, _NUM_BINS - 2)
      cp = jnp.minimum(ci, cv[pl.ds(jb, _LANES)])
      binv = jnp.where(cp > 0.0, binv, _NUM_BINS - 1)
      jvec = jb + iota
      for c in range(_TDIM):
        row = plsc.load_gather(tab, [binv, csplats[c]])
        plsc.store_scatter(obuf, [jvec, csplats[c]], row * cp)
      return 0

    lax.fori_loop(0, _GROUPS, group_step, 0)
    pltpu.sync_copy(obuf, out_h.at[b, i])
    return 0

  lax.fori_loop(0, _ROWS_PER_TILE, row_step, 0)


@jax.jit
def _encode(x, y, z, conf, w_flat, bias, gamma, beta):
  mesh = plsc.VectorSubcoreMesh(
      core_axis_name="c", subcore_axis_name="s",
      num_cores=_NUM_CORES, num_subcores=_NUM_SUBCORES)
  f = functools.partial(
      pl.kernel,
      out_type=jax.ShapeDtypeStruct((_B, _N, _N, _TDIM), jnp.float32),
      mesh=mesh,
      compiler_params=pltpu.CompilerParams(
          needs_layout_passes=False, use_tc_tiling_on_sc=False),
      scratch_types=[
          pltpu.VMEM((_N,), jnp.float32),
          pltpu.VMEM((_N,), jnp.float32),
          pltpu.VMEM((_N,), jnp.float32),
          pltpu.VMEM((_N,), jnp.float32),
          pltpu.VMEM((_TDIM * _NUM_BINS,), jnp.float32),
          pltpu.VMEM((_TDIM,), jnp.float32),
          pltpu.VMEM((_TDIM,), jnp.float32),
          pltpu.VMEM((_TDIM,), jnp.float32),
          pltpu.VMEM((_NUM_BINS, _TDIM), jnp.float32),
          pltpu.VMEM((_N, _TDIM), jnp.float32),
      ],
  )(_body)
  return f(x, y, z, conf, w_flat, bias, gamma, beta)


def kernel(template_coords, confidence, W, b, gamma, beta):
  x = template_coords[:, :, 0]
  y = template_coords[:, :, 1]
  z = template_coords[:, :, 2]
  return _encode(x, y, z, confidence, W.reshape(-1), b, gamma, beta)


# speedup vs baseline: 103.2156x; 1.3533x over previous
import functools

import jax
import jax.numpy as jnp
from jax import lax
from jax.experimental import pallas as pl
from jax.experimental.pallas import tpu as pltpu
from jax.experimental.pallas import tpu_sc as plsc

_B = 2
_N = 1024
_NUM_BINS = 22
_TDIM = 16
_MAX_DIST = 40.0
_INV_W = (_NUM_BINS - 1) / _MAX_DIST

_NUM_CORES = 2
_NUM_SUBCORES = 16
_LANES = 16
_NW = _NUM_CORES * _NUM_SUBCORES
_ROWS_PER_TILE = (_B * _N) // _NW
_GROUPS = _N // _LANES


def _rsqrt(v):
  i = lax.bitcast_convert_type(v, jnp.int32)
  i = 0x5F3759DF - lax.shift_right_arithmetic(i, 1)
  y = lax.bitcast_convert_type(i, jnp.float32)
  for _ in range(3):
    y = y * (1.5 - 0.5 * v * y * y)
  return y


def _body(x_h, y_h, z_h, c_h, w_h, b_h, g_h, bt_h, out_h,
          xv, yv, zv, cv, wv, bsv, gsv, btv, tab, obuf0, obuf1, sem0, sem1):
  cid = lax.axis_index("c")
  sid = lax.axis_index("s")
  wid = sid * _NUM_CORES + cid
  b = wid // (_NW // _B)
  row0 = (wid * _ROWS_PER_TILE) % _N

  pltpu.sync_copy(x_h.at[b], xv)
  pltpu.sync_copy(y_h.at[b], yv)
  pltpu.sync_copy(z_h.at[b], zv)
  pltpu.sync_copy(c_h.at[b], cv)
  pltpu.sync_copy(w_h, wv)
  pltpu.sync_copy(b_h, bsv)
  pltpu.sync_copy(g_h, gsv)
  pltpu.sync_copy(bt_h, btv)

  iota = lax.iota(jnp.int32, _LANES)
  iota_w = iota * _NUM_BINS

  bias = bsv[...]
  gamma = gsv[...]
  beta = btv[...]
  for k in range(_NUM_BINS):
    col = plsc.load_gather(wv, [iota_w + k])
    x = col + bias
    mean = jnp.sum(x) * (1.0 / _TDIM)
    e = x - mean
    var = jnp.sum(e * e) * (1.0 / _TDIM)
    r = _rsqrt(jnp.full((_LANES,), var + 1e-5, jnp.float32))
    tab[k, :] = jnp.maximum(e * r * gamma + beta, 0.0)

  csplats = [jnp.full((_LANES,), c, jnp.int32) for c in range(_TDIM)]

  def compute_row(i, ob):
    isplat = jnp.full((_LANES,), i, jnp.int32)
    xi = plsc.load_gather(xv, [isplat])
    yi = plsc.load_gather(yv, [isplat])
    zi = plsc.load_gather(zv, [isplat])
    ci = plsc.load_gather(cv, [isplat])

    @plsc.parallel_loop(0, _GROUPS, unroll=2)
    def group_step(g):
      jb = g * _LANES
      dx = xv[pl.ds(jb, _LANES)] - xi
      dy = yv[pl.ds(jb, _LANES)] - yi
      dz = zv[pl.ds(jb, _LANES)] - zi
      s = dx * dx + dy * dy + dz * dz + 1e-8
      d = s * _rsqrt(s)
      t = jnp.minimum(d * _INV_W, float(_NUM_BINS))
      ti = t.astype(jnp.int32)
      binv = jnp.where(ti.astype(jnp.float32) < t, ti + 1, ti)
      binv = jnp.minimum(binv, _NUM_BINS - 2)
      cp = jnp.minimum(ci, cv[pl.ds(jb, _LANES)])
      binv = jnp.where(cp > 0.0, binv, _NUM_BINS - 1)
      jvec = jb + iota
      for c in range(_TDIM):
        row = plsc.load_gather(tab, [binv, csplats[c]])
        plsc.store_scatter(ob, [jvec, csplats[c]], row * cp)

  def row_pair(kp, _):
    i0 = row0 + 2 * kp

    @pl.when(kp >= 1)
    def _drain0():
      pltpu.make_async_copy(out_h.at[b, i0], obuf0, sem0).wait()

    compute_row(i0, obuf0)
    pltpu.async_copy(obuf0, out_h.at[b, i0], sem0)

    i1 = i0 + 1

    @pl.when(kp >= 1)
    def _drain1():
      pltpu.make_async_copy(out_h.at[b, i1], obuf1, sem1).wait()

    compute_row(i1, obuf1)
    pltpu.async_copy(obuf1, out_h.at[b, i1], sem1)
    return 0

  lax.fori_loop(0, _ROWS_PER_TILE // 2, row_pair, 0)
  pltpu.make_async_copy(out_h.at[b, row0], obuf0, sem0).wait()
  pltpu.make_async_copy(out_h.at[b, row0], obuf1, sem1).wait()


@jax.jit
def _encode(x, y, z, conf, w_flat, bias, gamma, beta):
  mesh = plsc.VectorSubcoreMesh(
      core_axis_name="c", subcore_axis_name="s",
      num_cores=_NUM_CORES, num_subcores=_NUM_SUBCORES)
  f = functools.partial(
      pl.kernel,
      out_type=jax.ShapeDtypeStruct((_B, _N, _N, _TDIM), jnp.float32),
      mesh=mesh,
      compiler_params=pltpu.CompilerParams(
          needs_layout_passes=False, use_tc_tiling_on_sc=False),
      scratch_types=[
          pltpu.VMEM((_N,), jnp.float32),
          pltpu.VMEM((_N,), jnp.float32),
          pltpu.VMEM((_N,), jnp.float32),
          pltpu.VMEM((_N,), jnp.float32),
          pltpu.VMEM((_TDIM * _NUM_BINS,), jnp.float32),
          pltpu.VMEM((_TDIM,), jnp.float32),
          pltpu.VMEM((_TDIM,), jnp.float32),
          pltpu.VMEM((_TDIM,), jnp.float32),
          pltpu.VMEM((_NUM_BINS, _TDIM), jnp.float32),
          pltpu.VMEM((_N, _TDIM), jnp.float32),
          pltpu.VMEM((_N, _TDIM), jnp.float32),
          pltpu.SemaphoreType.DMA,
          pltpu.SemaphoreType.DMA,
      ],
  )(_body)
  return f(x, y, z, conf, w_flat, bias, gamma, beta)


def kernel(template_coords, confidence, W, b, gamma, beta):
  x = template_coords[:, :, 0]
  y = template_coords[:, :, 1]
  z = template_coords[:, :, 2]
  return _encode(x, y, z, confidence, W.reshape(-1), b, gamma, beta)


# speedup vs baseline: 115.3675x; 1.1177x over previous
import functools

import jax
import jax.numpy as jnp
from jax import lax
from jax.experimental import pallas as pl
from jax.experimental.pallas import tpu as pltpu
from jax.experimental.pallas import tpu_sc as plsc

_B = 2
_N = 1024
_NUM_BINS = 22
_TDIM = 16
_MAX_DIST = 40.0
_INV_W = (_NUM_BINS - 1) / _MAX_DIST

_NUM_CORES = 2
_NUM_SUBCORES = 16
_LANES = 16
_NW = _NUM_CORES * _NUM_SUBCORES
_ROWS_PER_TILE = (_B * _N) // _NW
_GROUPS = _N // _LANES


def _rsqrt(v):
  i = lax.bitcast_convert_type(v, jnp.int32)
  i = 0x5F3759DF - lax.shift_right_arithmetic(i, 1)
  y = lax.bitcast_convert_type(i, jnp.float32)
  for _ in range(3):
    y = y * (1.5 - 0.5 * v * y * y)
  return y


def _body(x_h, y_h, z_h, c_h, w_h, b_h, g_h, bt_h, out_h,
          xv, yv, zv, cv, wv, bsv, gsv, btv, tab, obuf0, obuf1, sem0, sem1):
  cid = lax.axis_index("c")
  sid = lax.axis_index("s")
  wid = sid * _NUM_CORES + cid
  b = wid // (_NW // _B)
  row0 = (wid * _ROWS_PER_TILE) % _N

  pltpu.sync_copy(x_h.at[b], xv)
  pltpu.sync_copy(y_h.at[b], yv)
  pltpu.sync_copy(z_h.at[b], zv)
  pltpu.sync_copy(c_h.at[b], cv)
  pltpu.sync_copy(w_h, wv)
  pltpu.sync_copy(b_h, bsv)
  pltpu.sync_copy(g_h, gsv)
  pltpu.sync_copy(bt_h, btv)

  iota = lax.iota(jnp.int32, _LANES)
  iota_w = iota * _NUM_BINS

  bias = bsv[...]
  gamma = gsv[...]
  beta = btv[...]
  for k in range(_NUM_BINS):
    col = plsc.load_gather(wv, [iota_w + k])
    x = col + bias
    mean = jnp.sum(x) * (1.0 / _TDIM)
    e = x - mean
    var = jnp.sum(e * e) * (1.0 / _TDIM)
    r = _rsqrt(jnp.full((_LANES,), var + 1e-5, jnp.float32))
    tab[k, :] = jnp.maximum(e * r * gamma + beta, 0.0)

  psplats = [jnp.full((_LANES,), p, jnp.int32) for p in range(_LANES)]

  def _xlane_splat(vec, p):
    return lax.gather(
        vec, psplats[p][:, None],
        lax.GatherDimensionNumbers(
            offset_dims=(), collapsed_slice_dims=(0,), start_index_map=(0,)),
        (1,), mode=lax.GatherScatterMode.PROMISE_IN_BOUNDS)

  def compute_row(i, ob):
    isplat = jnp.full((_LANES,), i, jnp.int32)
    xi = plsc.load_gather(xv, [isplat])
    yi = plsc.load_gather(yv, [isplat])
    zi = plsc.load_gather(zv, [isplat])
    ci = plsc.load_gather(cv, [isplat])

    @plsc.parallel_loop(0, _GROUPS, unroll=2)
    def group_step(g):
      jb = g * _LANES
      dx = xv[pl.ds(jb, _LANES)] - xi
      dy = yv[pl.ds(jb, _LANES)] - yi
      dz = zv[pl.ds(jb, _LANES)] - zi
      s = dx * dx + dy * dy + dz * dz + 1e-8
      d = s * _rsqrt(s)
      t = jnp.minimum(d * _INV_W, float(_NUM_BINS))
      ti = t.astype(jnp.int32)
      binv = jnp.where(ti.astype(jnp.float32) < t, ti + 1, ti)
      binv = jnp.minimum(binv, _NUM_BINS - 2)
      cp = jnp.minimum(ci, cv[pl.ds(jb, _LANES)])
      binv = jnp.where(cp > 0.0, binv, _NUM_BINS - 1)
      for p in range(_LANES):
        bsp = _xlane_splat(binv, p)
        csp = _xlane_splat(cp, p)
        row = plsc.load_gather(tab, [bsp, iota])
        ob[jb + p, :] = row * csp

  def row_pair(kp, _):
    i0 = row0 + 2 * kp

    @pl.when(kp >= 1)
    def _drain0():
      pltpu.make_async_copy(out_h.at[b, i0], obuf0, sem0).wait()

    compute_row(i0, obuf0)
    pltpu.async_copy(obuf0, out_h.at[b, i0], sem0)

    i1 = i0 + 1

    @pl.when(kp >= 1)
    def _drain1():
      pltpu.make_async_copy(out_h.at[b, i1], obuf1, sem1).wait()

    compute_row(i1, obuf1)
    pltpu.async_copy(obuf1, out_h.at[b, i1], sem1)
    return 0

  lax.fori_loop(0, _ROWS_PER_TILE // 2, row_pair, 0)
  pltpu.make_async_copy(out_h.at[b, row0], obuf0, sem0).wait()
  pltpu.make_async_copy(out_h.at[b, row0], obuf1, sem1).wait()


@jax.jit
def _encode(x, y, z, conf, w_flat, bias, gamma, beta):
  mesh = plsc.VectorSubcoreMesh(
      core_axis_name="c", subcore_axis_name="s",
      num_cores=_NUM_CORES, num_subcores=_NUM_SUBCORES)
  f = functools.partial(
      pl.kernel,
      out_type=jax.ShapeDtypeStruct((_B, _N, _N, _TDIM), jnp.float32),
      mesh=mesh,
      compiler_params=pltpu.CompilerParams(
          needs_layout_passes=False, use_tc_tiling_on_sc=False),
      scratch_types=[
          pltpu.VMEM((_N,), jnp.float32),
          pltpu.VMEM((_N,), jnp.float32),
          pltpu.VMEM((_N,), jnp.float32),
          pltpu.VMEM((_N,), jnp.float32),
          pltpu.VMEM((_TDIM * _NUM_BINS,), jnp.float32),
          pltpu.VMEM((_TDIM,), jnp.float32),
          pltpu.VMEM((_TDIM,), jnp.float32),
          pltpu.VMEM((_TDIM,), jnp.float32),
          pltpu.VMEM((_NUM_BINS, _TDIM), jnp.float32),
          pltpu.VMEM((_N, _TDIM), jnp.float32),
          pltpu.VMEM((_N, _TDIM), jnp.float32),
          pltpu.SemaphoreType.DMA,
          pltpu.SemaphoreType.DMA,
      ],
  )(_body)
  return f(x, y, z, conf, w_flat, bias, gamma, beta)


def kernel(template_coords, confidence, W, b, gamma, beta):
  x = template_coords[:, :, 0]
  y = template_coords[:, :, 1]
  z = template_coords[:, :, 2]
  return _encode(x, y, z, confidence, W.reshape(-1), b, gamma, beta)


# speedup vs baseline: 1164.8827x; 10.0972x over previous
import functools

import jax
import jax.numpy as jnp
from jax import lax
from jax.experimental import pallas as pl
from jax.experimental.pallas import tpu as pltpu
from jax.experimental.pallas import tpu_sc as plsc

_B = 2
_N = 1024
_NUM_BINS = 22
_TDIM = 16
_MAX_DIST = 40.0
_INV_W = (_NUM_BINS - 1) / _MAX_DIST

_NUM_CORES = 2
_NUM_SUBCORES = 16
_LANES = 16
_NW = _NUM_CORES * _NUM_SUBCORES
_ROWS_PER_TILE = (_B * _N) // _NW
_GROUPS = _N // _LANES


def _rsqrt(v, steps=3):
  i = lax.bitcast_convert_type(v, jnp.int32)
  i = 0x5F3759DF - lax.shift_right_arithmetic(i, 1)
  y = lax.bitcast_convert_type(i, jnp.float32)
  for _ in range(steps):
    y = y * (1.5 - 0.5 * v * y * y)
  return y


def _body(x_h, y_h, z_h, c_h, w_h, b_h, g_h, bt_h, out_h,
          xv, yv, zv, cv, wv, bsv, gsv, btv, tab, obuf0, obuf1, sem0, sem1):
  cid = lax.axis_index("c")
  sid = lax.axis_index("s")
  wid = sid * _NUM_CORES + cid
  b = wid // (_NW // _B)
  row0 = (wid * _ROWS_PER_TILE) % _N

  pltpu.sync_copy(x_h.at[b], xv)
  pltpu.sync_copy(y_h.at[b], yv)
  pltpu.sync_copy(z_h.at[b], zv)
  pltpu.sync_copy(c_h.at[b], cv)
  pltpu.sync_copy(w_h, wv)
  pltpu.sync_copy(b_h, bsv)
  pltpu.sync_copy(g_h, gsv)
  pltpu.sync_copy(bt_h, btv)

  iota = lax.iota(jnp.int32, _LANES)
  iota_w = iota * _NUM_BINS

  bias = bsv[...]
  gamma = gsv[...]
  beta = btv[...]
  for k in range(_NUM_BINS):
    col = plsc.load_gather(wv, [iota_w + k])
    x = col + bias
    mean = jnp.sum(x) * (1.0 / _TDIM)
    e = x - mean
    var = jnp.sum(e * e) * (1.0 / _TDIM)
    r = _rsqrt(jnp.full((_LANES,), var + 1e-5, jnp.float32))
    tab[pl.ds(17 * k, _TDIM)] = jnp.maximum(e * r * gamma + beta, 0.0)

  coff = [(c // 8) * 8192 + (c % 8) * 128 for c in range(_TDIM)]

  def compute_row(i, ob):
    isplat = jnp.full((_LANES,), i, jnp.int32)
    xi = plsc.load_gather(xv, [isplat])
    yi = plsc.load_gather(yv, [isplat])
    zi = plsc.load_gather(zv, [isplat])
    ci = plsc.load_gather(cv, [isplat])

    @plsc.parallel_loop(0, _GROUPS, unroll=1)
    def group_step(g):
      jb = g * _LANES
      dx = xv[pl.ds(jb, _LANES)] - xi
      dy = yv[pl.ds(jb, _LANES)] - yi
      dz = zv[pl.ds(jb, _LANES)] - zi
      s = dx * dx + dy * dy + dz * dz + 1e-8
      d = s * _rsqrt(s)
      t = jnp.minimum(d * _INV_W, float(_NUM_BINS))
      ti = t.astype(jnp.int32)
      binv = jnp.where(ti.astype(jnp.float32) < t, ti + 1, ti)
      binv = jnp.minimum(binv, _NUM_BINS - 2)
      cp = jnp.minimum(ci, cv[pl.ds(jb, _LANES)])
      binv = jnp.where(cp > 0.0, binv, _NUM_BINS - 1)
      obase = (g // 8) * 1024 + (g % 8) * _LANES
      tix = binv * 17
      for c in range(_TDIM):
        val = plsc.load_gather(tab, [tix + c if c else tix]) * cp
        ob[pl.ds(obase + coff[c], _LANES)] = val

  bufs = ((obuf0, sem0), (obuf1, sem1))

  def row_pair(kp, _):
    for r, (ob, sem) in enumerate(bufs):
      i = row0 + 2 * kp + r

      @pl.when(kp >= 1)
      def _drain(ob=ob, sem=sem, i=i):
        pltpu.make_async_copy(out_h.at[b, i], ob, sem).wait()

      compute_row(i, ob)
      pltpu.async_copy(ob, out_h.at[b, i], sem)
    return 0

  lax.fori_loop(0, _ROWS_PER_TILE // 2, row_pair, 0)
  for ob, sem in bufs:
    pltpu.make_async_copy(out_h.at[b, row0], ob, sem).wait()


@jax.jit
def _encode(x, y, z, conf, w_flat, bias, gamma, beta):
  mesh = plsc.VectorSubcoreMesh(
      core_axis_name="c", subcore_axis_name="s",
      num_cores=_NUM_CORES, num_subcores=_NUM_SUBCORES)
  f = functools.partial(
      pl.kernel,
      out_type=jax.ShapeDtypeStruct((_B, _N, _N * _TDIM), jnp.float32),
      mesh=mesh,
      compiler_params=pltpu.CompilerParams(
          needs_layout_passes=False, use_tc_tiling_on_sc=False),
      scratch_types=[
          pltpu.VMEM((_N,), jnp.float32),
          pltpu.VMEM((_N,), jnp.float32),
          pltpu.VMEM((_N,), jnp.float32),
          pltpu.VMEM((_N,), jnp.float32),
          pltpu.VMEM((_TDIM * _NUM_BINS,), jnp.float32),
          pltpu.VMEM((_TDIM,), jnp.float32),
          pltpu.VMEM((_TDIM,), jnp.float32),
          pltpu.VMEM((_TDIM,), jnp.float32),
          pltpu.VMEM((_NUM_BINS * 17,), jnp.float32),
          pltpu.VMEM((_N * _TDIM,), jnp.float32),
          pltpu.VMEM((_N * _TDIM,), jnp.float32),
          pltpu.SemaphoreType.DMA,
          pltpu.SemaphoreType.DMA,
      ],
  )(_body)
  return f(x, y, z, conf, w_flat, bias, gamma, beta)


def kernel(template_coords, confidence, W, b, gamma, beta):
  x = template_coords[:, :, 0]
  y = template_coords[:, :, 1]
  z = template_coords[:, :, 2]
  r = _encode(x, y, z, confidence, W.reshape(-1), b, gamma, beta)
  r6 = r.reshape(_B, _N, 2, 8, 8, 128)
  return r6.transpose(0, 1, 3, 5, 2, 4).reshape(_B, _N, _N, _TDIM)
